# baseline (device time: 103348 ns/iter reference)
import jax
import jax.numpy as jnp
from jax import lax
from jax.experimental import pallas as pl
from jax.experimental.pallas import tpu as pltpu

N_DEV = 4


def kernel(x, w_mat, scale_x, scale_w):
    m_total, _ = x.shape
    n = w_mat.shape[1]
    m_per = m_total // N_DEV
    n_half = n // 2

    def body(x_ref, w_ref, sx_ref, sw_ref, out_ref,
             w8_ref, send_r_ref, send_l_ref, recv_r_ref, recv_l_ref,
             send_sems_r, recv_sems_r, send_sems_l, recv_sems_l):
        my = lax.axis_index("i")
        left = lax.rem(my + N_DEV - 1, N_DEV)
        right = lax.rem(my + 1, N_DEV)

        barrier_sem = pltpu.get_barrier_semaphore()
        for nbr in (left, right):
            pl.semaphore_signal(barrier_sem, inc=1, device_id=(nbr,),
                                device_id_type=pl.DeviceIdType.MESH)
        pl.semaphore_wait(barrier_sem, 2)

        w8_ref[...] = w_ref[...].astype(jnp.bfloat16)

        def partial(c, lo):
            xs = x_ref[pl.ds(c * m_per, m_per), :].astype(jnp.bfloat16)
            ws = w8_ref[:, 0:n_half] if lo else w8_ref[:, n_half:n]
            return lax.dot_general(
                xs, ws,
                dimension_numbers=(((1,), (0,)), ((), ())),
                preferred_element_type=jnp.float32,
            )

        def silu_store(acc, lo):
            y = acc * (sx_ref[0] * sw_ref[0])
            o = y * (1.0 / (1.0 + jnp.exp(-y)))
            if lo:
                out_ref[:, 0:n_half] = o
            else:
                out_ref[:, n_half:n] = o

        send_r_ref[...] = partial(lax.rem(my + N_DEV - 1, N_DEV), True
                                  ).astype(jnp.bfloat16)
        send_l_ref[...] = partial(lax.rem(my + 1, N_DEV), False
                                  ).astype(jnp.bfloat16)
        for s in range(N_DEV - 1):
            rdma_r = pltpu.make_async_remote_copy(
                src_ref=send_r_ref,
                dst_ref=recv_r_ref.at[s],
                send_sem=send_sems_r.at[s],
                recv_sem=recv_sems_r.at[s],
                device_id=(right,),
                device_id_type=pl.DeviceIdType.MESH,
            )
            rdma_l = pltpu.make_async_remote_copy(
                src_ref=send_l_ref,
                dst_ref=recv_l_ref.at[s],
                send_sem=send_sems_l.at[s],
                recv_sem=recv_sems_l.at[s],
                device_id=(left,),
                device_id_type=pl.DeviceIdType.MESH,
            )
            rdma_r.start()
            rdma_l.start()
            if s < N_DEV - 2:
                nxt_r = partial(lax.rem(my + N_DEV - 2 - s, N_DEV), True)
                nxt_l = partial(lax.rem(my + 2 + s, N_DEV), False)
                rdma_r.wait()
                send_r_ref[...] = (recv_r_ref[s].astype(jnp.float32)
                                   + nxt_r).astype(jnp.bfloat16)
                rdma_l.wait()
                send_l_ref[...] = (recv_l_ref[s].astype(jnp.float32)
                                   + nxt_l).astype(jnp.bfloat16)
            else:
                own_r = partial(my, True)
                own_l = partial(my, False)
                rdma_r.wait()
                silu_store(recv_r_ref[s].astype(jnp.float32) + own_r, True)
                rdma_l.wait()
                silu_store(recv_l_ref[s].astype(jnp.float32) + own_l, False)

    return pl.pallas_call(
        body,
        out_shape=jax.ShapeDtypeStruct((m_per, n), jnp.float32),
        in_specs=[
            pl.BlockSpec(memory_space=pltpu.VMEM),
            pl.BlockSpec(memory_space=pltpu.VMEM),
            pl.BlockSpec(memory_space=pltpu.SMEM),
            pl.BlockSpec(memory_space=pltpu.SMEM),
        ],
        out_specs=pl.BlockSpec(memory_space=pltpu.VMEM),
        scratch_shapes=[
            pltpu.VMEM((x.shape[1], n), jnp.bfloat16),
            pltpu.VMEM((m_per, n_half), jnp.bfloat16),
            pltpu.VMEM((m_per, n_half), jnp.bfloat16),
            pltpu.VMEM((N_DEV - 1, m_per, n_half), jnp.bfloat16),
            pltpu.VMEM((N_DEV - 1, m_per, n_half), jnp.bfloat16),
            pltpu.SemaphoreType.DMA((N_DEV - 1,)),
            pltpu.SemaphoreType.DMA((N_DEV - 1,)),
            pltpu.SemaphoreType.DMA((N_DEV - 1,)),
            pltpu.SemaphoreType.DMA((N_DEV - 1,)),
        ],
        compiler_params=pltpu.CompilerParams(
            collective_id=0,
            vmem_limit_bytes=120 * 1024 * 1024,
        ),
    )(x, w_mat, scale_x, scale_w)


# device time: 101529 ns/iter; 1.0179x vs baseline; 1.0179x over previous
import jax
import jax.numpy as jnp
from jax import lax
from jax.experimental import pallas as pl
from jax.experimental.pallas import tpu as pltpu

N_DEV = 4


def kernel(x, w_mat, scale_x, scale_w):
    m_total, k = x.shape
    n = w_mat.shape[1]
    m_per = m_total // N_DEV
    n_half = n // 2

    def body(x_ref, w_ref, sx_ref, sw_ref, out_ref,
             x8_ref, w8_ref, send_r_ref, send_l_ref, recv_r_ref, recv_l_ref,
             send_sems_r, recv_sems_r, send_sems_l, recv_sems_l):
        my = lax.axis_index("i")
        left = lax.rem(my + N_DEV - 1, N_DEV)
        right = lax.rem(my + 1, N_DEV)

        barrier_sem = pltpu.get_barrier_semaphore()
        for nbr in (left, right):
            pl.semaphore_signal(barrier_sem, inc=1, device_id=(nbr,),
                                device_id_type=pl.DeviceIdType.MESH)
        pl.semaphore_wait(barrier_sem, 2)

        def cast_chunk(c):
            x8_ref[pl.ds(c * m_per, m_per), :] = x_ref[
                pl.ds(c * m_per, m_per), :].astype(jnp.float8_e4m3fn)

        def partial(c, lo):
            xs = x8_ref[pl.ds(c * m_per, m_per), :]
            ws = w8_ref[:, 0:n_half] if lo else w8_ref[:, n_half:n]
            return lax.dot_general(
                xs, ws,
                dimension_numbers=(((1,), (0,)), ((), ())),
                preferred_element_type=jnp.float32,
            )

        def rdma(dirn, s):
            if dirn == "r":
                return pltpu.make_async_remote_copy(
                    src_ref=send_r_ref, dst_ref=recv_r_ref.at[s],
                    send_sem=send_sems_r.at[s], recv_sem=recv_sems_r.at[s],
                    device_id=(right,), device_id_type=pl.DeviceIdType.MESH)
            return pltpu.make_async_remote_copy(
                src_ref=send_l_ref, dst_ref=recv_l_ref.at[s],
                send_sem=send_sems_l.at[s], recv_sem=recv_sems_l.at[s],
                device_id=(left,), device_id_type=pl.DeviceIdType.MESH)

        def silu_store(acc, lo):
            y = acc * (sx_ref[0] * sw_ref[0])
            o = y * (1.0 / (1.0 + jnp.exp(-y)))
            if lo:
                out_ref[:, 0:n_half] = o
            else:
                out_ref[:, n_half:n] = o

        c_m1 = lax.rem(my + N_DEV - 1, N_DEV)
        c_p1 = lax.rem(my + 1, N_DEV)
        c_p2 = lax.rem(my + 2, N_DEV)

        w8_ref[:, 0:n_half] = w_ref[:, 0:n_half].astype(jnp.float8_e4m3fn)
        cast_chunk(c_m1)
        send_r_ref[...] = partial(c_m1, True).astype(jnp.bfloat16)
        r0 = rdma("r", 0)
        r0.start()

        w8_ref[:, n_half:n] = w_ref[:, n_half:n].astype(jnp.float8_e4m3fn)
        cast_chunk(c_p1)
        send_l_ref[...] = partial(c_p1, False).astype(jnp.bfloat16)
        l0 = rdma("l", 0)
        l0.start()

        cast_chunk(c_p2)
        nxt_r = partial(c_p2, True)
        r0.wait()
        send_r_ref[...] = (recv_r_ref[0].astype(jnp.float32)
                           + nxt_r).astype(jnp.bfloat16)
        r1 = rdma("r", 1)
        r1.start()

        nxt_l = partial(c_p2, False)
        l0.wait()
        send_l_ref[...] = (recv_l_ref[0].astype(jnp.float32)
                           + nxt_l).astype(jnp.bfloat16)
        l1 = rdma("l", 1)
        l1.start()

        cast_chunk(my)
        nxt_r = partial(c_p1, True)
        r1.wait()
        send_r_ref[...] = (recv_r_ref[1].astype(jnp.float32)
                           + nxt_r).astype(jnp.bfloat16)
        r2 = rdma("r", 2)
        r2.start()

        nxt_l = partial(c_m1, False)
        l1.wait()
        send_l_ref[...] = (recv_l_ref[1].astype(jnp.float32)
                           + nxt_l).astype(jnp.bfloat16)
        l2 = rdma("l", 2)
        l2.start()

        own_r = partial(my, True)
        r2.wait()
        silu_store(recv_r_ref[2].astype(jnp.float32) + own_r, True)

        own_l = partial(my, False)
        l2.wait()
        silu_store(recv_l_ref[2].astype(jnp.float32) + own_l, False)

    return pl.pallas_call(
        body,
        out_shape=jax.ShapeDtypeStruct((m_per, n), jnp.float32),
        in_specs=[
            pl.BlockSpec(memory_space=pltpu.VMEM),
            pl.BlockSpec(memory_space=pltpu.VMEM),
            pl.BlockSpec(memory_space=pltpu.SMEM),
            pl.BlockSpec(memory_space=pltpu.SMEM),
        ],
        out_specs=pl.BlockSpec(memory_space=pltpu.VMEM),
        scratch_shapes=[
            pltpu.VMEM((m_total, k), jnp.float8_e4m3fn),
            pltpu.VMEM((k, n), jnp.float8_e4m3fn),
            pltpu.VMEM((m_per, n_half), jnp.bfloat16),
            pltpu.VMEM((m_per, n_half), jnp.bfloat16),
            pltpu.VMEM((N_DEV - 1, m_per, n_half), jnp.bfloat16),
            pltpu.VMEM((N_DEV - 1, m_per, n_half), jnp.bfloat16),
            pltpu.SemaphoreType.DMA((N_DEV - 1,)),
            pltpu.SemaphoreType.DMA((N_DEV - 1,)),
            pltpu.SemaphoreType.DMA((N_DEV - 1,)),
            pltpu.SemaphoreType.DMA((N_DEV - 1,)),
        ],
        compiler_params=pltpu.CompilerParams(
            collective_id=0,
            vmem_limit_bytes=120 * 1024 * 1024,
        ),
    )(x, w_mat, scale_x, scale_w)


# device time: 98459 ns/iter; 1.0497x vs baseline; 1.0312x over previous
import jax
import jax.numpy as jnp
from jax import lax
from jax.experimental import pallas as pl
from jax.experimental.pallas import tpu as pltpu

N_DEV = 4


def kernel(x, w_mat, scale_x, scale_w):
    m_total, k = x.shape
    n = w_mat.shape[1]
    m_per = m_total // N_DEV
    n_half = n // 2

    def body(x_ref, w_ref, sx_ref, sw_ref, out_ref,
             x8_ref, w8_ref, send_r_ref, send_l_ref, recv_r_ref, recv_l_ref,
             send_sems_r, recv_sems_r, send_sems_l, recv_sems_l):
        my = lax.axis_index("i")
        left = lax.rem(my + N_DEV - 1, N_DEV)
        right = lax.rem(my + 1, N_DEV)

        barrier_sem = pltpu.get_barrier_semaphore()
        for nbr in (left, right):
            pl.semaphore_signal(barrier_sem, inc=1, device_id=(nbr,),
                                device_id_type=pl.DeviceIdType.MESH)
        pl.semaphore_wait(barrier_sem, 2)

        def cast_chunk(c):
            x8_ref[pl.ds(c * m_per, m_per), :] = x_ref[
                pl.ds(c * m_per, m_per), :].astype(jnp.float8_e4m3fn)

        def partial(c, lo):
            xs = x8_ref[pl.ds(c * m_per, m_per), :]
            ws = w8_ref[:, 0:n_half] if lo else w8_ref[:, n_half:n]
            return lax.dot_general(
                xs, ws,
                dimension_numbers=(((1,), (0,)), ((), ())),
                preferred_element_type=jnp.float32,
            )

        def rdma(dirn, s):
            if dirn == "r":
                return pltpu.make_async_remote_copy(
                    src_ref=send_r_ref.at[s % 2], dst_ref=recv_r_ref.at[s],
                    send_sem=send_sems_r.at[s], recv_sem=recv_sems_r.at[s],
                    device_id=(right,), device_id_type=pl.DeviceIdType.MESH)
            return pltpu.make_async_remote_copy(
                src_ref=send_l_ref.at[s % 2], dst_ref=recv_l_ref.at[s],
                send_sem=send_sems_l.at[s], recv_sem=recv_sems_l.at[s],
                device_id=(left,), device_id_type=pl.DeviceIdType.MESH)

        def silu_store(acc, lo):
            y = acc * (sx_ref[0] * sw_ref[0])
            o = y * (1.0 / (1.0 + jnp.exp(-y)))
            if lo:
                out_ref[:, 0:n_half] = o
            else:
                out_ref[:, n_half:n] = o

        c_m1 = lax.rem(my + N_DEV - 1, N_DEV)
        c_p1 = lax.rem(my + 1, N_DEV)
        c_p2 = lax.rem(my + 2, N_DEV)

        w8_ref[:, 0:n_half] = w_ref[:, 0:n_half].astype(jnp.float8_e4m3fn)
        cast_chunk(c_m1)
        send_r_ref[0] = partial(c_m1, True).astype(jnp.bfloat16)
        r0 = rdma("r", 0)
        r0.start()

        w8_ref[:, n_half:n] = w_ref[:, n_half:n].astype(jnp.float8_e4m3fn)
        cast_chunk(c_p1)
        send_l_ref[0] = partial(c_p1, False).astype(jnp.bfloat16)
        l0 = rdma("l", 0)
        l0.start()

        cast_chunk(c_p2)
        send_r_ref[1] = partial(c_p2, True).astype(jnp.bfloat16)
        send_l_ref[1] = partial(c_p2, False).astype(jnp.bfloat16)
        r0.wait()
        send_r_ref[1] = send_r_ref[1] + recv_r_ref[0]
        r1 = rdma("r", 1)
        r1.start()
        l0.wait()
        send_l_ref[1] = send_l_ref[1] + recv_l_ref[0]
        l1 = rdma("l", 1)
        l1.start()

        cast_chunk(my)
        send_r_ref[0] = partial(c_p1, True).astype(jnp.bfloat16)
        send_l_ref[0] = partial(c_m1, False).astype(jnp.bfloat16)
        r1.wait()
        send_r_ref[0] = send_r_ref[0] + recv_r_ref[1]
        r2 = rdma("r", 2)
        r2.start()
        l1.wait()
        send_l_ref[0] = send_l_ref[0] + recv_l_ref[1]
        l2 = rdma("l", 2)
        l2.start()

        send_r_ref[1] = partial(my, True).astype(jnp.bfloat16)
        send_l_ref[1] = partial(my, False).astype(jnp.bfloat16)
        r2.wait()
        silu_store(send_r_ref[1].astype(jnp.float32)
                   + recv_r_ref[2].astype(jnp.float32), True)
        l2.wait()
        silu_store(send_l_ref[1].astype(jnp.float32)
                   + recv_l_ref[2].astype(jnp.float32), False)

    return pl.pallas_call(
        body,
        out_shape=jax.ShapeDtypeStruct((m_per, n), jnp.float32),
        in_specs=[
            pl.BlockSpec(memory_space=pltpu.VMEM),
            pl.BlockSpec(memory_space=pltpu.VMEM),
            pl.BlockSpec(memory_space=pltpu.SMEM),
            pl.BlockSpec(memory_space=pltpu.SMEM),
        ],
        out_specs=pl.BlockSpec(memory_space=pltpu.VMEM),
        scratch_shapes=[
            pltpu.VMEM((m_total, k), jnp.float8_e4m3fn),
            pltpu.VMEM((k, n), jnp.float8_e4m3fn),
            pltpu.VMEM((2, m_per, n_half), jnp.bfloat16),
            pltpu.VMEM((2, m_per, n_half), jnp.bfloat16),
            pltpu.VMEM((N_DEV - 1, m_per, n_half), jnp.bfloat16),
            pltpu.VMEM((N_DEV - 1, m_per, n_half), jnp.bfloat16),
            pltpu.SemaphoreType.DMA((N_DEV - 1,)),
            pltpu.SemaphoreType.DMA((N_DEV - 1,)),
            pltpu.SemaphoreType.DMA((N_DEV - 1,)),
            pltpu.SemaphoreType.DMA((N_DEV - 1,)),
        ],
        compiler_params=pltpu.CompilerParams(
            collective_id=0,
            vmem_limit_bytes=120 * 1024 * 1024,
        ),
    )(x, w_mat, scale_x, scale_w)


# device time: 92719 ns/iter; 1.1146x vs baseline; 1.0619x over previous
import jax
import jax.numpy as jnp
from jax import lax
from jax.experimental import pallas as pl
from jax.experimental.pallas import tpu as pltpu

N_DEV = 4
N_SUB = 2


def kernel(x, w_mat, scale_x, scale_w):
    m_total, k = x.shape
    n = w_mat.shape[1]
    m_per = m_total // N_DEV
    n_half = n // 2
    m_sub = m_per // N_SUB

    def body(x_ref, w_ref, sx_ref, sw_ref, out_ref,
             x8_ref, w8_ref, send_r_ref, send_l_ref, recv_r_ref, recv_l_ref,
             send_sems_r, recv_sems_r, send_sems_l, recv_sems_l):
        my = lax.axis_index("i")
        left = lax.rem(my + N_DEV - 1, N_DEV)
        right = lax.rem(my + 1, N_DEV)

        barrier_sem = pltpu.get_barrier_semaphore()
        for nbr in (left, right):
            pl.semaphore_signal(barrier_sem, inc=1, device_id=(nbr,),
                                device_id_type=pl.DeviceIdType.MESH)
        pl.semaphore_wait(barrier_sem, 2)

        def cast_rows(c, b0, nb):
            r0 = c * m_per + b0 * m_sub
            x8_ref[pl.ds(r0, nb * m_sub), :] = x_ref[
                pl.ds(r0, nb * m_sub), :].astype(jnp.float8_e4m3fn)

        def partial(c, lo, b0, nb):
            xs = x8_ref[pl.ds(c * m_per + b0 * m_sub, nb * m_sub), :]
            ws = w8_ref[:, 0:n_half] if lo else w8_ref[:, n_half:n]
            return lax.dot_general(
                xs, ws,
                dimension_numbers=(((1,), (0,)), ((), ())),
                preferred_element_type=jnp.float32,
            )

        def rdma(dirn, s, b):
            rows = pl.ds(b * m_sub, m_sub)
            if dirn == "r":
                return pltpu.make_async_remote_copy(
                    src_ref=send_r_ref.at[s % 2, rows],
                    dst_ref=recv_r_ref.at[s, rows],
                    send_sem=send_sems_r.at[s, b],
                    recv_sem=recv_sems_r.at[s, b],
                    device_id=(right,), device_id_type=pl.DeviceIdType.MESH)
            return pltpu.make_async_remote_copy(
                src_ref=send_l_ref.at[s % 2, rows],
                dst_ref=recv_l_ref.at[s, rows],
                send_sem=send_sems_l.at[s, b],
                recv_sem=recv_sems_l.at[s, b],
                device_id=(left,), device_id_type=pl.DeviceIdType.MESH)

        def silu_store(acc, lo, b):
            y = acc * (sx_ref[0] * sw_ref[0])
            o = y * (1.0 / (1.0 + jnp.exp(-y)))
            rows = pl.ds(b * m_sub, m_sub)
            if lo:
                out_ref[rows, 0:n_half] = o
            else:
                out_ref[rows, n_half:n] = o

        c_m1 = lax.rem(my + N_DEV - 1, N_DEV)
        c_p1 = lax.rem(my + 1, N_DEV)
        c_p2 = lax.rem(my + 2, N_DEV)

        w8_ref[:, 0:n_half] = w_ref[:, 0:n_half].astype(jnp.float8_e4m3fn)
        cast_rows(c_m1, 0, 1)
        send_r_ref[0, pl.ds(0, m_sub)] = partial(
            c_m1, True, 0, 1).astype(jnp.bfloat16)
        r0a = rdma("r", 0, 0)
        r0a.start()

        w8_ref[:, n_half:n] = w_ref[:, n_half:n].astype(jnp.float8_e4m3fn)
        cast_rows(c_p1, 0, 1)
        send_l_ref[0, pl.ds(0, m_sub)] = partial(
            c_p1, False, 0, 1).astype(jnp.bfloat16)
        l0a = rdma("l", 0, 0)
        l0a.start()

        cast_rows(c_m1, 1, 1)
        send_r_ref[0, pl.ds(m_sub, m_sub)] = partial(
            c_m1, True, 1, 1).astype(jnp.bfloat16)
        r0b = rdma("r", 0, 1)
        r0b.start()

        cast_rows(c_p1, 1, 1)
        send_l_ref[0, pl.ds(m_sub, m_sub)] = partial(
            c_p1, False, 1, 1).astype(jnp.bfloat16)
        l0b = rdma("l", 0, 1)
        l0b.start()

        cast_rows(c_p2, 0, N_SUB)
        send_r_ref[1] = partial(c_p2, True, 0, N_SUB).astype(jnp.bfloat16)
        send_l_ref[1] = partial(c_p2, False, 0, N_SUB).astype(jnp.bfloat16)

        sub = pl.ds(0, m_sub)
        sub_b = pl.ds(m_sub, m_sub)
        r0a.wait()
        send_r_ref[1, sub] = send_r_ref[1, sub] + recv_r_ref[0, sub]
        r1a = rdma("r", 1, 0)
        r1a.start()
        l0a.wait()
        send_l_ref[1, sub] = send_l_ref[1, sub] + recv_l_ref[0, sub]
        l1a = rdma("l", 1, 0)
        l1a.start()
        r0b.wait()
        send_r_ref[1, sub_b] = send_r_ref[1, sub_b] + recv_r_ref[0, sub_b]
        r1b = rdma("r", 1, 1)
        r1b.start()
        l0b.wait()
        send_l_ref[1, sub_b] = send_l_ref[1, sub_b] + recv_l_ref[0, sub_b]
        l1b = rdma("l", 1, 1)
        l1b.start()

        cast_rows(my, 0, N_SUB)
        send_r_ref[0] = partial(c_p1, True, 0, N_SUB).astype(jnp.bfloat16)
        send_l_ref[0] = partial(c_m1, False, 0, N_SUB).astype(jnp.bfloat16)

        r1a.wait()
        send_r_ref[0, sub] = send_r_ref[0, sub] + recv_r_ref[1, sub]
        r2a = rdma("r", 2, 0)
        r2a.start()
        l1a.wait()
        send_l_ref[0, sub] = send_l_ref[0, sub] + recv_l_ref[1, sub]
        l2a = rdma("l", 2, 0)
        l2a.start()
        r1b.wait()
        send_r_ref[0, sub_b] = send_r_ref[0, sub_b] + recv_r_ref[1, sub_b]
        r2b = rdma("r", 2, 1)
        r2b.start()
        l1b.wait()
        send_l_ref[0, sub_b] = send_l_ref[0, sub_b] + recv_l_ref[1, sub_b]
        l2b = rdma("l", 2, 1)
        l2b.start()

        send_r_ref[1] = partial(my, True, 0, N_SUB).astype(jnp.bfloat16)
        send_l_ref[1] = partial(my, False, 0, N_SUB).astype(jnp.bfloat16)

        r2a.wait()
        silu_store(send_r_ref[1, sub].astype(jnp.float32)
                   + recv_r_ref[2, sub].astype(jnp.float32), True, 0)
        l2a.wait()
        silu_store(send_l_ref[1, sub].astype(jnp.float32)
                   + recv_l_ref[2, sub].astype(jnp.float32), False, 0)
        r2b.wait()
        silu_store(send_r_ref[1, sub_b].astype(jnp.float32)
                   + recv_r_ref[2, sub_b].astype(jnp.float32), True, 1)
        l2b.wait()
        silu_store(send_l_ref[1, sub_b].astype(jnp.float32)
                   + recv_l_ref[2, sub_b].astype(jnp.float32), False, 1)

    return pl.pallas_call(
        body,
        out_shape=jax.ShapeDtypeStruct((m_per, n), jnp.float32),
        in_specs=[
            pl.BlockSpec(memory_space=pltpu.VMEM),
            pl.BlockSpec(memory_space=pltpu.VMEM),
            pl.BlockSpec(memory_space=pltpu.SMEM),
            pl.BlockSpec(memory_space=pltpu.SMEM),
        ],
        out_specs=pl.BlockSpec(memory_space=pltpu.VMEM),
        scratch_shapes=[
            pltpu.VMEM((m_total, k), jnp.float8_e4m3fn),
            pltpu.VMEM((k, n), jnp.float8_e4m3fn),
            pltpu.VMEM((2, m_per, n_half), jnp.bfloat16),
            pltpu.VMEM((2, m_per, n_half), jnp.bfloat16),
            pltpu.VMEM((N_DEV - 1, m_per, n_half), jnp.bfloat16),
            pltpu.VMEM((N_DEV - 1, m_per, n_half), jnp.bfloat16),
            pltpu.SemaphoreType.DMA((N_DEV - 1, N_SUB)),
            pltpu.SemaphoreType.DMA((N_DEV - 1, N_SUB)),
            pltpu.SemaphoreType.DMA((N_DEV - 1, N_SUB)),
            pltpu.SemaphoreType.DMA((N_DEV - 1, N_SUB)),
        ],
        compiler_params=pltpu.CompilerParams(
            collective_id=0,
            vmem_limit_bytes=120 * 1024 * 1024,
        ),
    )(x, w_mat, scale_x, scale_w)


# device time: 92077 ns/iter; 1.1224x vs baseline; 1.0070x over previous
import jax
import jax.numpy as jnp
from jax import lax
from jax.experimental import pallas as pl
from jax.experimental.pallas import tpu as pltpu

N_DEV = 4
N_SUB = 4


def kernel(x, w_mat, scale_x, scale_w):
    m_total, k = x.shape
    n = w_mat.shape[1]
    m_per = m_total // N_DEV
    n_half = n // 2
    m_sub = m_per // N_SUB

    def body(x_ref, w_ref, sx_ref, sw_ref, out_ref,
             x8_ref, w8_ref, send_r_ref, send_l_ref, recv_r_ref, recv_l_ref,
             send_sems_r, recv_sems_r, send_sems_l, recv_sems_l):
        my = lax.axis_index("i")
        left = lax.rem(my + N_DEV - 1, N_DEV)
        right = lax.rem(my + 1, N_DEV)

        barrier_sem = pltpu.get_barrier_semaphore()
        for nbr in (left, right):
            pl.semaphore_signal(barrier_sem, inc=1, device_id=(nbr,),
                                device_id_type=pl.DeviceIdType.MESH)
        pl.semaphore_wait(barrier_sem, 2)

        def rows(b):
            return pl.ds(b * m_sub, m_sub)

        def cast_rows(c, b0, nb):
            r0 = c * m_per + b0 * m_sub
            x8_ref[pl.ds(r0, nb * m_sub), :] = x_ref[
                pl.ds(r0, nb * m_sub), :].astype(jnp.float8_e4m3fn)

        def partial(c, lo, b0, nb):
            xs = x8_ref[pl.ds(c * m_per + b0 * m_sub, nb * m_sub), :]
            ws = w8_ref[:, 0:n_half] if lo else w8_ref[:, n_half:n]
            return lax.dot_general(
                xs, ws,
                dimension_numbers=(((1,), (0,)), ((), ())),
                preferred_element_type=jnp.float32,
            )

        def rdma(dirn, s, b):
            if dirn == "r":
                return pltpu.make_async_remote_copy(
                    src_ref=send_r_ref.at[s % 2, rows(b)],
                    dst_ref=recv_r_ref.at[s, rows(b)],
                    send_sem=send_sems_r.at[s, b],
                    recv_sem=recv_sems_r.at[s, b],
                    device_id=(right,), device_id_type=pl.DeviceIdType.MESH)
            return pltpu.make_async_remote_copy(
                src_ref=send_l_ref.at[s % 2, rows(b)],
                dst_ref=recv_l_ref.at[s, rows(b)],
                send_sem=send_sems_l.at[s, b],
                recv_sem=recv_sems_l.at[s, b],
                device_id=(left,), device_id_type=pl.DeviceIdType.MESH)

        def silu_store(acc, lo, b):
            y = acc * (sx_ref[0] * sw_ref[0])
            o = y * (1.0 / (1.0 + jnp.exp(-y)))
            if lo:
                out_ref[rows(b), 0:n_half] = o
            else:
                out_ref[rows(b), n_half:n] = o

        c_m1 = lax.rem(my + N_DEV - 1, N_DEV)
        c_p1 = lax.rem(my + 1, N_DEV)
        c_p2 = lax.rem(my + 2, N_DEV)

        prev_r, prev_l = [], []
        w8_ref[:, 0:n_half] = w_ref[:, 0:n_half].astype(jnp.float8_e4m3fn)
        for b in range(N_SUB):
            cast_rows(c_m1, b, 1)
            send_r_ref[0, rows(b)] = partial(c_m1, True, b, 1).astype(
                jnp.bfloat16)
            d = rdma("r", 0, b)
            d.start()
            prev_r.append(d)
            if b == 0:
                w8_ref[:, n_half:n] = w_ref[:, n_half:n].astype(
                    jnp.float8_e4m3fn)
            cast_rows(c_p1, b, 1)
            send_l_ref[0, rows(b)] = partial(c_p1, False, b, 1).astype(
                jnp.bfloat16)
            d = rdma("l", 0, b)
            d.start()
            prev_l.append(d)

        for s in (1, 2):
            slot = s % 2
            if s == 1:
                cast_rows(c_p2, 0, N_SUB)
                cr, cl = c_p2, c_p2
            else:
                cast_rows(my, 0, N_SUB)
                cr, cl = c_p1, c_m1
            send_r_ref[slot] = partial(cr, True, 0, N_SUB).astype(
                jnp.bfloat16)
            send_l_ref[slot] = partial(cl, False, 0, N_SUB).astype(
                jnp.bfloat16)
            cur_r, cur_l = [], []
            for b in range(N_SUB):
                prev_r[b].wait()
                send_r_ref[slot, rows(b)] = (send_r_ref[slot, rows(b)]
                                             + recv_r_ref[s - 1, rows(b)])
                d = rdma("r", s, b)
                d.start()
                cur_r.append(d)
                prev_l[b].wait()
                send_l_ref[slot, rows(b)] = (send_l_ref[slot, rows(b)]
                                             + recv_l_ref[s - 1, rows(b)])
                d = rdma("l", s, b)
                d.start()
                cur_l.append(d)
            prev_r, prev_l = cur_r, cur_l

        send_r_ref[1] = partial(my, True, 0, N_SUB).astype(jnp.bfloat16)
        send_l_ref[1] = partial(my, False, 0, N_SUB).astype(jnp.bfloat16)
        for b in range(N_SUB):
            prev_r[b].wait()
            silu_store(send_r_ref[1, rows(b)].astype(jnp.float32)
                       + recv_r_ref[2, rows(b)].astype(jnp.float32), True, b)
            prev_l[b].wait()
            silu_store(send_l_ref[1, rows(b)].astype(jnp.float32)
                       + recv_l_ref[2, rows(b)].astype(jnp.float32), False, b)

    return pl.pallas_call(
        body,
        out_shape=jax.ShapeDtypeStruct((m_per, n), jnp.float32),
        in_specs=[
            pl.BlockSpec(memory_space=pltpu.VMEM),
            pl.BlockSpec(memory_space=pltpu.VMEM),
            pl.BlockSpec(memory_space=pltpu.SMEM),
            pl.BlockSpec(memory_space=pltpu.SMEM),
        ],
        out_specs=pl.BlockSpec(memory_space=pltpu.VMEM),
        scratch_shapes=[
            pltpu.VMEM((m_total, k), jnp.float8_e4m3fn),
            pltpu.VMEM((k, n), jnp.float8_e4m3fn),
            pltpu.VMEM((2, m_per, n_half), jnp.bfloat16),
            pltpu.VMEM((2, m_per, n_half), jnp.bfloat16),
            pltpu.VMEM((N_DEV - 1, m_per, n_half), jnp.bfloat16),
            pltpu.VMEM((N_DEV - 1, m_per, n_half), jnp.bfloat16),
            pltpu.SemaphoreType.DMA((N_DEV - 1, N_SUB)),
            pltpu.SemaphoreType.DMA((N_DEV - 1, N_SUB)),
            pltpu.SemaphoreType.DMA((N_DEV - 1, N_SUB)),
            pltpu.SemaphoreType.DMA((N_DEV - 1, N_SUB)),
        ],
        compiler_params=pltpu.CompilerParams(
            collective_id=0,
            vmem_limit_bytes=120 * 1024 * 1024,
        ),
    )(x, w_mat, scale_x, scale_w)


# device time: 91501 ns/iter; 1.1295x vs baseline; 1.0063x over previous
import jax
import jax.numpy as jnp
from jax import lax
from jax.experimental import pallas as pl
from jax.experimental.pallas import tpu as pltpu

N_DEV = 4
N_SUB = 4


def kernel(x, w_mat, scale_x, scale_w):
    m_total, k = x.shape
    n = w_mat.shape[1]
    m_per = m_total // N_DEV
    n_half = n // 2
    m_sub = m_per // N_SUB

    def body(x_ref, w_ref, sx_ref, sw_ref, out_ref,
             x8_ref, w8_ref, send_r_ref, send_l_ref, recv_r_ref, recv_l_ref,
             send_sems_r, recv_sems_r, send_sems_l, recv_sems_l):
        my = lax.axis_index("i")
        left = lax.rem(my + N_DEV - 1, N_DEV)
        right = lax.rem(my + 1, N_DEV)

        barrier_sem = pltpu.get_barrier_semaphore()
        for nbr in (left, right):
            pl.semaphore_signal(barrier_sem, inc=1, device_id=(nbr,),
                                device_id_type=pl.DeviceIdType.MESH)

        def rows(b):
            return pl.ds(b * m_sub, m_sub)

        def cast_rows(c, b0, nb):
            r0 = c * m_per + b0 * m_sub
            x8_ref[pl.ds(r0, nb * m_sub), :] = x_ref[
                pl.ds(r0, nb * m_sub), :].astype(jnp.float8_e4m3fn)

        def partial(c, lo, b0, nb):
            xs = x8_ref[pl.ds(c * m_per + b0 * m_sub, nb * m_sub), :]
            ws = w8_ref[:, 0:n_half] if lo else w8_ref[:, n_half:n]
            return lax.dot_general(
                xs, ws,
                dimension_numbers=(((1,), (0,)), ((), ())),
                preferred_element_type=jnp.float32,
            )

        def rdma(dirn, s, b):
            if dirn == "r":
                return pltpu.make_async_remote_copy(
                    src_ref=send_r_ref.at[s % 2, rows(b)],
                    dst_ref=recv_r_ref.at[s, rows(b)],
                    send_sem=send_sems_r.at[s, b],
                    recv_sem=recv_sems_r.at[s, b],
                    device_id=(right,), device_id_type=pl.DeviceIdType.MESH)
            return pltpu.make_async_remote_copy(
                src_ref=send_l_ref.at[s % 2, rows(b)],
                dst_ref=recv_l_ref.at[s, rows(b)],
                send_sem=send_sems_l.at[s, b],
                recv_sem=recv_sems_l.at[s, b],
                device_id=(left,), device_id_type=pl.DeviceIdType.MESH)

        def silu_store(acc, lo, b):
            y = acc * (sx_ref[0] * sw_ref[0])
            o = y * (1.0 / (1.0 + jnp.exp(-y)))
            if lo:
                out_ref[rows(b), 0:n_half] = o
            else:
                out_ref[rows(b), n_half:n] = o

        c_m1 = lax.rem(my + N_DEV - 1, N_DEV)
        c_p1 = lax.rem(my + 1, N_DEV)
        c_p2 = lax.rem(my + 2, N_DEV)

        prev_r, prev_l = [], []
        w8_ref[:, 0:n_half] = w_ref[:, 0:n_half].astype(jnp.float8_e4m3fn)
        for b in range(N_SUB):
            cast_rows(c_m1, b, 1)
            send_r_ref[0, rows(b)] = partial(c_m1, True, b, 1).astype(
                jnp.bfloat16)
            if b == 0:
                pl.semaphore_wait(barrier_sem, 2)
            d = rdma("r", 0, b)
            d.start()
            prev_r.append(d)
            if b == 0:
                w8_ref[:, n_half:n] = w_ref[:, n_half:n].astype(
                    jnp.float8_e4m3fn)
            cast_rows(c_p1, b, 1)
            send_l_ref[0, rows(b)] = partial(c_p1, False, b, 1).astype(
                jnp.bfloat16)
            d = rdma("l", 0, b)
            d.start()
            prev_l.append(d)

        for s in (1, 2):
            slot = s % 2
            if s == 1:
                cast_rows(c_p2, 0, N_SUB)
                cr, cl = c_p2, c_p2
            else:
                cast_rows(my, 0, N_SUB)
                cr, cl = c_p1, c_m1
            send_r_ref[slot] = partial(cr, True, 0, N_SUB).astype(
                jnp.bfloat16)
            send_l_ref[slot] = partial(cl, False, 0, N_SUB).astype(
                jnp.bfloat16)
            cur_r, cur_l = [], []
            for b in range(N_SUB):
                prev_r[b].wait()
                send_r_ref[slot, rows(b)] = (send_r_ref[slot, rows(b)]
                                             + recv_r_ref[s - 1, rows(b)])
                d = rdma("r", s, b)
                d.start()
                cur_r.append(d)
                prev_l[b].wait()
                send_l_ref[slot, rows(b)] = (send_l_ref[slot, rows(b)]
                                             + recv_l_ref[s - 1, rows(b)])
                d = rdma("l", s, b)
                d.start()
                cur_l.append(d)
            prev_r, prev_l = cur_r, cur_l

        send_r_ref[1] = partial(my, True, 0, N_SUB).astype(jnp.bfloat16)
        send_l_ref[1] = partial(my, False, 0, N_SUB).astype(jnp.bfloat16)
        for b in range(N_SUB):
            prev_r[b].wait()
            silu_store(send_r_ref[1, rows(b)].astype(jnp.float32)
                       + recv_r_ref[2, rows(b)].astype(jnp.float32), True, b)
            prev_l[b].wait()
            silu_store(send_l_ref[1, rows(b)].astype(jnp.float32)
                       + recv_l_ref[2, rows(b)].astype(jnp.float32), False, b)

    return pl.pallas_call(
        body,
        out_shape=jax.ShapeDtypeStruct((m_per, n), jnp.float32),
        in_specs=[
            pl.BlockSpec(memory_space=pltpu.VMEM),
            pl.BlockSpec(memory_space=pltpu.VMEM),
            pl.BlockSpec(memory_space=pltpu.SMEM),
            pl.BlockSpec(memory_space=pltpu.SMEM),
        ],
        out_specs=pl.BlockSpec(memory_space=pltpu.VMEM),
        scratch_shapes=[
            pltpu.VMEM((m_total, k), jnp.float8_e4m3fn),
            pltpu.VMEM((k, n), jnp.float8_e4m3fn),
            pltpu.VMEM((2, m_per, n_half), jnp.bfloat16),
            pltpu.VMEM((2, m_per, n_half), jnp.bfloat16),
            pltpu.VMEM((N_DEV - 1, m_per, n_half), jnp.bfloat16),
            pltpu.VMEM((N_DEV - 1, m_per, n_half), jnp.bfloat16),
            pltpu.SemaphoreType.DMA((N_DEV - 1, N_SUB)),
            pltpu.SemaphoreType.DMA((N_DEV - 1, N_SUB)),
            pltpu.SemaphoreType.DMA((N_DEV - 1, N_SUB)),
            pltpu.SemaphoreType.DMA((N_DEV - 1, N_SUB)),
        ],
        compiler_params=pltpu.CompilerParams(
            collective_id=0,
            vmem_limit_bytes=120 * 1024 * 1024,
        ),
    )(x, w_mat, scale_x, scale_w)
